# baseline (speedup 1.0000x reference)
import functools
import math

import numpy as np
import jax
import jax.numpy as jnp
from jax import lax
from jax.experimental import pallas as pl

_B = 2
_L = 2048
_DM = 1024
_H = 16
_DH = 64
_K = int(math.floor(math.log(_L)))
_NF = 1024
_NH = 1024

_PREC_DFT = lax.Precision.HIGHEST
_PREC_PROJ = lax.Precision.DEFAULT
_PREC_MIX = lax.Precision.HIGHEST


def _dft_consts():
    l = np.arange(_NH, dtype=np.float64)
    k = np.arange(_NF, dtype=np.float64)
    ang = 2.0 * np.pi * np.outer(k, l) / _L
    fce = np.cos(ang)
    fce[:, 0] *= 0.5
    fse = -np.sin(ang)
    n = np.arange(_NH, dtype=np.float64)
    angi = 2.0 * np.pi * np.outer(n, k) / _L
    ck = np.where(np.arange(_NF) == 0, 1.0, 2.0) / _L
    ace = np.cos(angi) * ck[None, :]
    ase = -np.sin(angi) * ck[None, :]
    fnq = np.zeros((8, _L))
    fnq[0] = np.where(np.arange(_L) % 2 == 0, 1.0, -1.0)
    return (jnp.asarray(fce, jnp.float32), jnp.asarray(fse, jnp.float32),
            jnp.asarray(ace, jnp.float32), jnp.asarray(ase, jnp.float32),
            jnp.asarray(fnq, jnp.float32))


def _mm_kernel(x_ref, w_ref, o_ref):
    o_ref[...] = jnp.dot(x_ref[...], w_ref[...],
                         preferred_element_type=jnp.float32,
                         precision=_PREC_PROJ)


def _matmul(x, w, bm=512):
    m, kk = x.shape
    n = w.shape[1]
    return pl.pallas_call(
        _mm_kernel,
        grid=(m // bm,),
        in_specs=[pl.BlockSpec((bm, kk), lambda i: (i, 0)),
                  pl.BlockSpec((kk, n), lambda i: (0, 0))],
        out_specs=pl.BlockSpec((bm, n), lambda i: (i, 0)),
        out_shape=jax.ShapeDtypeStruct((m, n), jnp.float32),
    )(x, w)


def _fold(x, xr):
    e = x[:_NH, :] + xr[:_NH, :]
    o = x[:_NH, :] - xr[:_NH, :]
    return e, o, x[_NH:_NH + 1, :]


def _spectrum_kernel(fce_ref, fse_ref, fnq_ref, qp_ref, kp_ref,
                     qx_ref, kx_ref, pr_ref, pi_ref, pn_ref, *, bk):
    j = pl.program_id(2)
    fc = fce_ref[...]
    fs = fse_ref[...]
    qp = qp_ref[0]
    kp = kp_ref[0]
    eq, oq, q1024 = _fold(qp, qx_ref[0])
    ek, ok, k1024 = _fold(kp, kx_ref[0])
    kg = j * bk + lax.broadcasted_iota(jnp.int32, (bk, 1), 0)
    alt = jnp.where(kg % 2 == 0, jnp.float32(1.0), jnp.float32(-1.0))
    qr = jnp.dot(fc, eq, preferred_element_type=jnp.float32,
                 precision=_PREC_DFT) + alt * q1024
    qi = jnp.dot(fs, oq, preferred_element_type=jnp.float32,
                 precision=_PREC_DFT)
    kr = jnp.dot(fc, ek, preferred_element_type=jnp.float32,
                 precision=_PREC_DFT) + alt * k1024
    ki = jnp.dot(fs, ok, preferred_element_type=jnp.float32,
                 precision=_PREC_DFT)
    pr_ref[0] = qr * kr + qi * ki
    pi_ref[0] = qi * kr - qr * ki

    @pl.when(j == 0)
    def _():
        fn = fnq_ref[...]
        qn = jnp.dot(fn, qp, preferred_element_type=jnp.float32,
                     precision=_PREC_DFT)
        kn = jnp.dot(fn, kp, preferred_element_type=jnp.float32,
                     precision=_PREC_DFT)
        pn_ref[0] = qn * kn * jnp.float32(1.0 / _L)


def _spectrum(fce, fse, fnq, qp, kp, qx, kx, bk=512, bc=256):
    return pl.pallas_call(
        functools.partial(_spectrum_kernel, bk=bk),
        grid=(_B, _DM // bc, _NF // bk),
        in_specs=[pl.BlockSpec((bk, _NH), lambda b, c, j: (j, 0)),
                  pl.BlockSpec((bk, _NH), lambda b, c, j: (j, 0)),
                  pl.BlockSpec((8, _L), lambda b, c, j: (0, 0)),
                  pl.BlockSpec((1, _L, bc), lambda b, c, j: (b, 0, c)),
                  pl.BlockSpec((1, _L, bc), lambda b, c, j: (b, 0, c)),
                  pl.BlockSpec((1, _L, bc), lambda b, c, j: (b, 0, c)),
                  pl.BlockSpec((1, _L, bc), lambda b, c, j: (b, 0, c))],
        out_specs=[pl.BlockSpec((1, bk, bc), lambda b, c, j: (b, j, c)),
                   pl.BlockSpec((1, bk, bc), lambda b, c, j: (b, j, c)),
                   pl.BlockSpec((1, 8, bc), lambda b, c, j: (b, 0, c))],
        out_shape=[jax.ShapeDtypeStruct((_B, _NF, _DM), jnp.float32),
                   jax.ShapeDtypeStruct((_B, _NF, _DM), jnp.float32),
                   jax.ShapeDtypeStruct((_B, 8, _DM), jnp.float32)],
    )(fce, fse, fnq, qp, kp, qx, kx)


def _corr_kernel(ace_ref, ase_ref, pr_ref, pi_ref, pn_ref,
                 top_ref, brev_ref, *, bl):
    i = pl.program_id(1)
    pr = pr_ref[0]
    pi = pi_ref[0]
    cc = jnp.dot(ace_ref[...], pr,
                 preferred_element_type=jnp.float32, precision=_PREC_DFT)
    ss = jnp.dot(ase_ref[...], pi,
                 preferred_element_type=jnp.float32, precision=_PREC_DFT)
    n_iota = i * bl + lax.broadcasted_iota(jnp.int32, (bl, 1), 0)
    sign = jnp.where(n_iota % 2 == 0, jnp.float32(1.0), jnp.float32(-1.0))
    cc = cc + sign * pn_ref[0, 0:1, :]
    top_ref[0] = cc + ss
    kf = lax.broadcasted_iota(jnp.int32, (_NF, 1), 0)
    w24 = jnp.where(kf == 0, jnp.float32(1.0), jnp.float32(2.0)) / _L
    w24 = jnp.where(kf % 2 == 0, w24, -w24)
    c1024 = jnp.sum(pr * w24, axis=0, keepdims=True) + pn_ref[0, 0:1, :]
    brev = cc - ss
    brev_ref[0] = jnp.where(n_iota == 0, c1024, brev)


def _corr(ace, ase, pr, pi, pn, bl=256, bc=512):
    return pl.pallas_call(
        functools.partial(_corr_kernel, bl=bl),
        grid=(_B, _NH // bl, _DM // bc),
        in_specs=[pl.BlockSpec((bl, _NF), lambda b, i, c: (i, 0)),
                  pl.BlockSpec((bl, _NF), lambda b, i, c: (i, 0)),
                  pl.BlockSpec((1, _NF, bc), lambda b, i, c: (b, 0, c)),
                  pl.BlockSpec((1, _NF, bc), lambda b, i, c: (b, 0, c)),
                  pl.BlockSpec((1, 8, bc), lambda b, i, c: (b, 0, c))],
        out_specs=[pl.BlockSpec((1, bl, bc), lambda b, i, c: (b, i, c)),
                   pl.BlockSpec((1, bl, bc), lambda b, i, c: (b, i, c))],
        out_shape=[jax.ShapeDtypeStruct((_B, _NH, _DM), jnp.float32),
                   jax.ShapeDtypeStruct((_B, _NH, _DM), jnp.float32)],
    )(ace, ase, pr, pi, pn)


def _topk_kernel(top_ref, brev_ref, mt_ref, *, bc):
    j = pl.program_id(1)
    cc = jnp.concatenate([top_ref[0], brev_ref[0]], axis=0)
    half = lax.broadcasted_iota(jnp.int32, (_NH, 1), 0)
    li_top = half
    li_brev = jnp.where(half == 0, _NH, 2 * _NH - half)
    li = jnp.concatenate([li_top, li_brev], axis=0)
    li = jnp.broadcast_to(li, cc.shape).astype(jnp.int32)
    neg = jnp.float32(-3.0e38)
    vals = []
    taus = []
    c = cc
    for _ in range(_K):
        m = jnp.max(c, axis=0, keepdims=True)
        idx = jnp.min(jnp.where(c == m, li, _L), axis=0, keepdims=True)
        vals.append(m)
        taus.append(idx)
        c = jnp.where(li == idx, neg, c)
    v = jnp.concatenate(vals, axis=0)
    tau = jnp.concatenate(taus, axis=0)
    v = v - jnp.max(v, axis=0, keepdims=True)
    e = jnp.exp(v)
    w = e / jnp.sum(e, axis=0, keepdims=True)
    tch = j * bc + lax.broadcasted_iota(jnp.int32, (1, bc), 1)
    head_base = (tch // _DH) * _DH
    tloc = tch % _DH
    iota_s = lax.broadcasted_iota(jnp.int32, (_DM, bc), 0)
    mt = jnp.zeros((_DM, bc), jnp.float32)
    for i in range(_K):
        src = head_base + lax.rem(tloc - tau[i:i + 1, :] + _L * _DH, _DH)
        mt = mt + jnp.where(iota_s == src, w[i:i + 1, :], 0.0)
    mt_ref[0] = mt


def _topk(top, brev, bc=256):
    return pl.pallas_call(
        functools.partial(_topk_kernel, bc=bc),
        grid=(_B, _DM // bc),
        in_specs=[pl.BlockSpec((1, _NH, bc), lambda b, j: (b, 0, j)),
                  pl.BlockSpec((1, _NH, bc), lambda b, j: (b, 0, j))],
        out_specs=pl.BlockSpec((1, _DM, bc), lambda b, j: (b, 0, j)),
        out_shape=jax.ShapeDtypeStruct((_B, _DM, _DM), jnp.float32),
    )(top, brev)


def _mix_kernel(vp_ref, mt_ref, o_ref):
    o_ref[0] = jnp.dot(vp_ref[0], mt_ref[0],
                       preferred_element_type=jnp.float32,
                       precision=_PREC_MIX)


def _mix(vp, mt, bl=512):
    return pl.pallas_call(
        _mix_kernel,
        grid=(_B, _L // bl),
        in_specs=[pl.BlockSpec((1, bl, _DM), lambda b, i: (b, i, 0)),
                  pl.BlockSpec((1, _DM, _DM), lambda b, i: (b, 0, 0))],
        out_specs=pl.BlockSpec((1, bl, _DM), lambda b, i: (b, i, 0)),
        out_shape=jax.ShapeDtypeStruct((_B, _L, _DM), jnp.float32),
    )(vp, mt)


def kernel(queries, keys, values, wq, wk, wv, wo):
    fce, fse, ace, ase, fnq = _dft_consts()
    q2 = queries.reshape(_B * _L, _DM)
    k2 = keys.reshape(_B * _L, _DM)
    v2 = values.reshape(_B * _L, _DM)
    qp = _matmul(q2, wq).reshape(_B, _L, _DM)
    kp = _matmul(k2, wk).reshape(_B, _L, _DM)
    vp = _matmul(v2, wv).reshape(_B, _L, _DM)
    qx = jnp.roll(jnp.flip(qp, axis=1), 1, axis=1)
    kx = jnp.roll(jnp.flip(kp, axis=1), 1, axis=1)
    pr, pi, pn = _spectrum(fce, fse, fnq, qp, kp, qx, kx)
    top, brev = _corr(ace, ase, pr, pi, pn)
    mt = _topk(top, brev)
    oc = _mix(vp, mt)
    r = oc.reshape(_B, 2, _DM, _H, _DH).transpose(0, 4, 3, 1, 2)
    r = r.reshape(_B * _L, _DM)
    out = _matmul(r, wo)
    return out.reshape(_B, _L, _DM)


# speedup vs baseline: 1.3672x; 1.3672x over previous
import functools
import math

import numpy as np
import jax
import jax.numpy as jnp
from jax import lax
from jax.experimental import pallas as pl

_B = 2
_L = 2048
_DM = 1024
_H = 16
_DH = 64
_K = int(math.floor(math.log(_L)))
_NF = 1024
_NH = 1024

_PREC_DFT = lax.Precision.HIGHEST
_PREC_PROJ = lax.Precision.DEFAULT
_PREC_MIX = lax.Precision.HIGHEST


def _dft_consts():
    l = np.arange(_L, dtype=np.float64)
    k = np.arange(_NF, dtype=np.float64)
    ang = 2.0 * np.pi * np.outer(k, l) / _L
    fce = np.cos(ang)
    fse = -np.sin(ang)
    n = np.arange(_NH, dtype=np.float64)
    angi = 2.0 * np.pi * np.outer(n, k) / _L
    ck = np.where(np.arange(_NF) == 0, 1.0, 2.0) / _L
    ace = np.cos(angi) * ck[None, :]
    ase = -np.sin(angi) * ck[None, :]
    fnq = np.zeros((8, _L))
    fnq[0] = np.where(np.arange(_L) % 2 == 0, 1.0, -1.0)
    return (jnp.asarray(fce, jnp.float32), jnp.asarray(fse, jnp.float32),
            jnp.asarray(ace, jnp.float32), jnp.asarray(ase, jnp.float32),
            jnp.asarray(fnq, jnp.float32))


def _mm_kernel(x_ref, w_ref, o_ref):
    o_ref[...] = jnp.dot(x_ref[...], w_ref[...],
                         preferred_element_type=jnp.float32,
                         precision=_PREC_PROJ)


def _matmul(x, w, bm=512):
    m, kk = x.shape
    n = w.shape[1]
    return pl.pallas_call(
        _mm_kernel,
        grid=(m // bm,),
        in_specs=[pl.BlockSpec((bm, kk), lambda i: (i, 0)),
                  pl.BlockSpec((kk, n), lambda i: (0, 0))],
        out_specs=pl.BlockSpec((bm, n), lambda i: (i, 0)),
        out_shape=jax.ShapeDtypeStruct((m, n), jnp.float32),
    )(x, w)


def _spectrum_kernel(fce_ref, fse_ref, fnq_ref, qp_ref, kp_ref,
                     pr_ref, pi_ref, pn_ref, *, bk):
    j = pl.program_id(2)
    fc = fce_ref[...]
    fs = fse_ref[...]
    qp = qp_ref[0]
    kp = kp_ref[0]
    qr = jnp.dot(fc, qp, preferred_element_type=jnp.float32,
                 precision=_PREC_DFT)
    qi = jnp.dot(fs, qp, preferred_element_type=jnp.float32,
                 precision=_PREC_DFT)
    kr = jnp.dot(fc, kp, preferred_element_type=jnp.float32,
                 precision=_PREC_DFT)
    ki = jnp.dot(fs, kp, preferred_element_type=jnp.float32,
                 precision=_PREC_DFT)
    pr_ref[0] = qr * kr + qi * ki
    pi_ref[0] = qi * kr - qr * ki

    @pl.when(j == 0)
    def _():
        fn = fnq_ref[...]
        qn = jnp.dot(fn, qp, preferred_element_type=jnp.float32,
                     precision=_PREC_DFT)
        kn = jnp.dot(fn, kp, preferred_element_type=jnp.float32,
                     precision=_PREC_DFT)
        pn_ref[0] = qn * kn * jnp.float32(1.0 / _L)


def _spectrum(fce, fse, fnq, qp, kp, bk=256, bc=512):
    return pl.pallas_call(
        functools.partial(_spectrum_kernel, bk=bk),
        grid=(_B, _DM // bc, _NF // bk),
        in_specs=[pl.BlockSpec((bk, _L), lambda b, c, j: (j, 0)),
                  pl.BlockSpec((bk, _L), lambda b, c, j: (j, 0)),
                  pl.BlockSpec((8, _L), lambda b, c, j: (0, 0)),
                  pl.BlockSpec((1, _L, bc), lambda b, c, j: (b, 0, c)),
                  pl.BlockSpec((1, _L, bc), lambda b, c, j: (b, 0, c))],
        out_specs=[pl.BlockSpec((1, bk, bc), lambda b, c, j: (b, j, c)),
                   pl.BlockSpec((1, bk, bc), lambda b, c, j: (b, j, c)),
                   pl.BlockSpec((1, 8, bc), lambda b, c, j: (b, 0, c))],
        out_shape=[jax.ShapeDtypeStruct((_B, _NF, _DM), jnp.float32),
                   jax.ShapeDtypeStruct((_B, _NF, _DM), jnp.float32),
                   jax.ShapeDtypeStruct((_B, 8, _DM), jnp.float32)],
    )(fce, fse, fnq, qp, kp)


def _corr_kernel(ace_ref, ase_ref, pr_ref, pi_ref, pn_ref,
                 top_ref, brev_ref, *, bl):
    i = pl.program_id(1)
    pr = pr_ref[0]
    pi = pi_ref[0]
    cc = jnp.dot(ace_ref[...], pr,
                 preferred_element_type=jnp.float32, precision=_PREC_DFT)
    ss = jnp.dot(ase_ref[...], pi,
                 preferred_element_type=jnp.float32, precision=_PREC_DFT)
    n_iota = i * bl + lax.broadcasted_iota(jnp.int32, (bl, 1), 0)
    sign = jnp.where(n_iota % 2 == 0, jnp.float32(1.0), jnp.float32(-1.0))
    cc = cc + sign * pn_ref[0, 0:1, :]
    top_ref[0] = cc + ss
    kf = lax.broadcasted_iota(jnp.int32, (_NF, 1), 0)
    w24 = jnp.where(kf == 0, jnp.float32(1.0), jnp.float32(2.0)) / _L
    w24 = jnp.where(kf % 2 == 0, w24, -w24)
    c1024 = jnp.sum(pr * w24, axis=0, keepdims=True) + pn_ref[0, 0:1, :]
    brev = cc - ss
    brev_ref[0] = jnp.where(n_iota == 0, c1024, brev)


def _corr(ace, ase, pr, pi, pn, bl=256, bc=512):
    return pl.pallas_call(
        functools.partial(_corr_kernel, bl=bl),
        grid=(_B, _NH // bl, _DM // bc),
        in_specs=[pl.BlockSpec((bl, _NF), lambda b, i, c: (i, 0)),
                  pl.BlockSpec((bl, _NF), lambda b, i, c: (i, 0)),
                  pl.BlockSpec((1, _NF, bc), lambda b, i, c: (b, 0, c)),
                  pl.BlockSpec((1, _NF, bc), lambda b, i, c: (b, 0, c)),
                  pl.BlockSpec((1, 8, bc), lambda b, i, c: (b, 0, c))],
        out_specs=[pl.BlockSpec((1, bl, bc), lambda b, i, c: (b, i, c)),
                   pl.BlockSpec((1, bl, bc), lambda b, i, c: (b, i, c))],
        out_shape=[jax.ShapeDtypeStruct((_B, _NH, _DM), jnp.float32),
                   jax.ShapeDtypeStruct((_B, _NH, _DM), jnp.float32)],
    )(ace, ase, pr, pi, pn)


def _topk_kernel(top_ref, brev_ref, mt_ref, *, bc):
    j = pl.program_id(1)
    cc = jnp.concatenate([top_ref[0], brev_ref[0]], axis=0)
    half = lax.broadcasted_iota(jnp.int32, (_NH, 1), 0)
    li_top = half
    li_brev = jnp.where(half == 0, _NH, 2 * _NH - half)
    li = jnp.concatenate([li_top, li_brev], axis=0)
    li = jnp.broadcast_to(li, cc.shape).astype(jnp.int32)
    neg = jnp.float32(-3.0e38)
    vals = []
    taus = []
    c = cc
    for _ in range(_K):
        m = jnp.max(c, axis=0, keepdims=True)
        idx = jnp.min(jnp.where(c == m, li, _L), axis=0, keepdims=True)
        vals.append(m)
        taus.append(idx)
        c = jnp.where(li == idx, neg, c)
    v = jnp.concatenate(vals, axis=0)
    tau = jnp.concatenate(taus, axis=0)
    v = v - jnp.max(v, axis=0, keepdims=True)
    e = jnp.exp(v)
    w = e / jnp.sum(e, axis=0, keepdims=True)
    tch = j * bc + lax.broadcasted_iota(jnp.int32, (1, bc), 1)
    head_base = (tch // _DH) * _DH
    tloc = tch % _DH
    iota_s = lax.broadcasted_iota(jnp.int32, (_DM, bc), 0)
    mt = jnp.zeros((_DM, bc), jnp.float32)
    for i in range(_K):
        src = head_base + lax.rem(tloc - tau[i:i + 1, :] + _L * _DH, _DH)
        mt = mt + jnp.where(iota_s == src, w[i:i + 1, :], 0.0)
    mt_ref[0] = mt


def _topk(top, brev, bc=256):
    return pl.pallas_call(
        functools.partial(_topk_kernel, bc=bc),
        grid=(_B, _DM // bc),
        in_specs=[pl.BlockSpec((1, _NH, bc), lambda b, j: (b, 0, j)),
                  pl.BlockSpec((1, _NH, bc), lambda b, j: (b, 0, j))],
        out_specs=pl.BlockSpec((1, _DM, bc), lambda b, j: (b, 0, j)),
        out_shape=jax.ShapeDtypeStruct((_B, _DM, _DM), jnp.float32),
    )(top, brev)


def _mix_kernel(vp_ref, mt_ref, o_ref):
    o_ref[0] = jnp.dot(vp_ref[0], mt_ref[0],
                       preferred_element_type=jnp.float32,
                       precision=_PREC_MIX)


def _mix(vp, mt, bl=512):
    return pl.pallas_call(
        _mix_kernel,
        grid=(_B, _L // bl),
        in_specs=[pl.BlockSpec((1, bl, _DM), lambda b, i: (b, i, 0)),
                  pl.BlockSpec((1, _DM, _DM), lambda b, i: (b, 0, 0))],
        out_specs=pl.BlockSpec((1, bl, _DM), lambda b, i: (b, i, 0)),
        out_shape=jax.ShapeDtypeStruct((_B, _L, _DM), jnp.float32),
    )(vp, mt)


def kernel(queries, keys, values, wq, wk, wv, wo):
    fce, fse, ace, ase, fnq = _dft_consts()
    q2 = queries.reshape(_B * _L, _DM)
    k2 = keys.reshape(_B * _L, _DM)
    v2 = values.reshape(_B * _L, _DM)
    qp = _matmul(q2, wq).reshape(_B, _L, _DM)
    kp = _matmul(k2, wk).reshape(_B, _L, _DM)
    vp = _matmul(v2, wv).reshape(_B, _L, _DM)
    pr, pi, pn = _spectrum(fce, fse, fnq, qp, kp)
    top, brev = _corr(ace, ase, pr, pi, pn)
    mt = _topk(top, brev)
    oc = _mix(vp, mt)
    r = oc.reshape(_B, 2, _DM, _H, _DH).transpose(0, 4, 3, 1, 2)
    r = r.reshape(_B * _L, _DM)
    out = _matmul(r, wo)
    return out.reshape(_B, _L, _DM)
